# Initial kernel scaffold; baseline (speedup 1.0000x reference)
import jax
import jax.numpy as jnp
from jax import lax
from jax.experimental import pallas as pl
from jax.experimental.pallas import tpu as pltpu
from jax.experimental.pallas import tpu_sc as plsc

N_NODES = 10000
N_EDGES = 320000
D_FEAT = 128

NC = 2
NS = 16
NW = NC * NS
EPW = N_EDGES // NW
C = 80
NCHUNK = EPW // C
G = C // 16


def _body(c_hbm, g_hbm, src_hbm, dst_hbm, out_hbm,
          idx_u, idx_v, u_rows, v_rows, out_buf, sem_u, sem_v):
    wid = lax.axis_index("s") * NC + lax.axis_index("c")
    base = wid * EPW
    lane = jnp.arange(16, dtype=jnp.int32)

    def chunk_body(ci, _):
        eb = base + ci * C
        pltpu.sync_copy(src_hbm.at[pl.ds(eb, C)], idx_u)
        pltpu.sync_copy(dst_hbm.at[pl.ds(eb, C)], idx_v)
        cu = pltpu.async_copy(c_hbm.at[idx_u], u_rows, sem_u)
        cv = pltpu.async_copy(g_hbm.at[idx_v], v_rows, sem_v)
        cu.wait()
        cv.wait()

        def group_body(gi, _):
            rows = gi * 16 + lane

            def dot_body(d, acc):
                col = jnp.full((16,), d, dtype=jnp.int32)
                hu = plsc.load_gather(u_rows, [rows, col])
                hv = plsc.load_gather(v_rows, [rows, col])
                return acc + hu * hv

            acc = lax.fori_loop(0, D_FEAT, dot_body,
                                jnp.zeros((16,), jnp.float32))
            pred = 1.0 / (1.0 + jnp.exp(-acc))
            out_buf[pl.ds(gi * 16, 16)] = pred
            return 0

        lax.fori_loop(0, G, group_body, 0)
        pltpu.sync_copy(out_buf, out_hbm.at[pl.ds(eb, C)])
        return 0

    lax.fori_loop(0, NCHUNK, chunk_body, 0)


@jax.jit
def _decode(c_feat, g_feat, src, dst):
    mesh = plsc.VectorSubcoreMesh(core_axis_name="c", subcore_axis_name="s",
                                  num_cores=NC, num_subcores=NS)
    return pl.kernel(
        _body,
        out_type=jax.ShapeDtypeStruct((N_EDGES,), jnp.float32),
        mesh=mesh,
        scratch_types=[
            pltpu.VMEM((C,), jnp.int32),
            pltpu.VMEM((C,), jnp.int32),
            pltpu.VMEM((C, D_FEAT), jnp.float32),
            pltpu.VMEM((C, D_FEAT), jnp.float32),
            pltpu.VMEM((C,), jnp.float32),
            pltpu.SemaphoreType.DMA,
            pltpu.SemaphoreType.DMA,
        ],
    )(c_feat, g_feat, src, dst)


def kernel(c_feat, g_feat, edge_index):
    ei = edge_index.astype(jnp.int32)
    out = _decode(c_feat, g_feat, ei[0], ei[1])
    return out.reshape(N_EDGES, 1)


# speedup vs baseline: 1.0963x; 1.0963x over previous
import jax
import jax.numpy as jnp
from jax import lax
from jax.experimental import pallas as pl
from jax.experimental.pallas import tpu as pltpu
from jax.experimental.pallas import tpu_sc as plsc

N_NODES = 10000
N_EDGES = 320000
D_FEAT = 128

NC = 2
NS = 16
NW = NC * NS
EPW = N_EDGES // NW
C = 80
NCHUNK = EPW // C
G = C // 16


def _body(c_hbm, g_hbm, src_hbm, dst_hbm, out_hbm,
          idx_u, idx_v, u_rows, v_rows, out_buf, sem_u, sem_v):
    wid = lax.axis_index("s") * NC + lax.axis_index("c")
    base = wid * EPW
    lane = jnp.arange(16, dtype=jnp.int32)

    def chunk_body(ci, _):
        eb = base + ci * C
        pltpu.sync_copy(src_hbm.at[pl.ds(eb, C)], idx_u)
        pltpu.sync_copy(dst_hbm.at[pl.ds(eb, C)], idx_v)
        cu = pltpu.async_copy(c_hbm.at[idx_u], u_rows, sem_u)
        cv = pltpu.async_copy(g_hbm.at[idx_v], v_rows, sem_v)
        cu.wait()
        cv.wait()

        def group_body(gi, _):
            rows = gi * 16 + lane

            def dot_body(d, acc):
                col = jnp.full((16,), d, dtype=jnp.int32)
                hu = plsc.load_gather(u_rows, [rows, col])
                hv = plsc.load_gather(v_rows, [rows, col])
                return acc + hu * hv

            acc = lax.fori_loop(0, D_FEAT, dot_body,
                                jnp.zeros((16,), jnp.float32))
            pred = 1.0 / (1.0 + jnp.exp(-acc))
            out_buf[pl.ds(gi * 16, 16)] = pred
            return 0

        lax.fori_loop(0, G, group_body, 0)
        pltpu.sync_copy(out_buf, out_hbm.at[pl.ds(eb, C)])
        return 0

    lax.fori_loop(0, NCHUNK, chunk_body, 0)


@jax.jit
def _decode(c_feat, g_feat, src, dst):
    mesh = plsc.VectorSubcoreMesh(core_axis_name="c", subcore_axis_name="s",
                                  num_cores=NC, num_subcores=NS)
    return pl.kernel(
        _body,
        out_type=jax.ShapeDtypeStruct((N_EDGES,), jnp.float32),
        mesh=mesh,
        compiler_params=pltpu.CompilerParams(needs_layout_passes=False),
        scratch_types=[
            pltpu.VMEM((C,), jnp.int32),
            pltpu.VMEM((C,), jnp.int32),
            pltpu.VMEM((C, D_FEAT), jnp.float32),
            pltpu.VMEM((C, D_FEAT), jnp.float32),
            pltpu.VMEM((C,), jnp.float32),
            pltpu.SemaphoreType.DMA,
            pltpu.SemaphoreType.DMA,
        ],
    )(c_feat, g_feat, src, dst)


def kernel(c_feat, g_feat, edge_index):
    ei = edge_index.astype(jnp.int32)
    out = _decode(c_feat, g_feat, ei[0], ei[1])
    return out.reshape(N_EDGES, 1)


# speedup vs baseline: 1.1652x; 1.0628x over previous
import jax
import jax.numpy as jnp
from jax import lax
from jax.experimental import pallas as pl
from jax.experimental.pallas import tpu as pltpu
from jax.experimental.pallas import tpu_sc as plsc

N_NODES = 10000
N_EDGES = 320000
D_FEAT = 128

NC = 2
NS = 16
NW = NC * NS
EPW = N_EDGES // NW
C = 80
NCHUNK = EPW // C
G = C // 16


def _body(c_hbm, g_hbm, src_hbm, dst_hbm, out_hbm,
          idx_u, idx_v, u0, u1, v0, v1, out_all,
          sem_u0, sem_u1, sem_v0, sem_v1):
    wid = lax.axis_index("s") * NC + lax.axis_index("c")
    base = wid * EPW
    lane = jnp.arange(16, dtype=jnp.int32)

    pltpu.sync_copy(src_hbm.at[pl.ds(base, EPW)], idx_u)
    pltpu.sync_copy(dst_hbm.at[pl.ds(base, EPW)], idx_v)

    def fire(ci, ub, vb, su, sv):
        off = ci * C
        pltpu.async_copy(c_hbm.at[idx_u.at[pl.ds(off, C)]], ub, su)
        pltpu.async_copy(g_hbm.at[idx_v.at[pl.ds(off, C)]], vb, sv)

    def wait(ub, vb, su, sv):
        pltpu.make_async_copy(c_hbm.at[idx_u.at[pl.ds(0, C)]], ub, su).wait()
        pltpu.make_async_copy(g_hbm.at[idx_v.at[pl.ds(0, C)]], vb, sv).wait()

    def compute(ci, ub, vb):
        def group_body(gi, _):
            rows = gi * 16 + lane
            a0 = jnp.zeros((16,), jnp.float32)
            a1 = jnp.zeros((16,), jnp.float32)
            a2 = jnp.zeros((16,), jnp.float32)
            a3 = jnp.zeros((16,), jnp.float32)
            for d in range(0, D_FEAT, 4):
                for k in range(4):
                    col = jnp.full((16,), d + k, dtype=jnp.int32)
                    hu = plsc.load_gather(ub, [rows, col])
                    hv = plsc.load_gather(vb, [rows, col])
                    if k == 0:
                        a0 = a0 + hu * hv
                    elif k == 1:
                        a1 = a1 + hu * hv
                    elif k == 2:
                        a2 = a2 + hu * hv
                    else:
                        a3 = a3 + hu * hv
            acc = (a0 + a1) + (a2 + a3)
            pred = 1.0 / (1.0 + jnp.exp(-acc))
            out_all[pl.ds(ci * C + gi * 16, 16)] = pred
            return 0

        lax.fori_loop(0, G, group_body, 0)

    fire(0, u0, v0, sem_u0, sem_v0)

    def pair_body(i2, _):
        ci0 = i2 * 2
        ci1 = ci0 + 1
        fire(ci1, u1, v1, sem_u1, sem_v1)
        wait(u0, v0, sem_u0, sem_v0)
        compute(ci0, u0, v0)
        fire(ci0 + 2, u0, v0, sem_u0, sem_v0)
        wait(u1, v1, sem_u1, sem_v1)
        compute(ci1, u1, v1)
        return 0

    lax.fori_loop(0, (NCHUNK - 1) // 2, pair_body, 0)
    wait(u0, v0, sem_u0, sem_v0)
    compute(NCHUNK - 1, u0, v0)

    pltpu.sync_copy(out_all, out_hbm.at[pl.ds(base, EPW)])


@jax.jit
def _decode(c_feat, g_feat, src, dst):
    mesh = plsc.VectorSubcoreMesh(core_axis_name="c", subcore_axis_name="s",
                                  num_cores=NC, num_subcores=NS)
    return pl.kernel(
        _body,
        out_type=jax.ShapeDtypeStruct((N_EDGES,), jnp.float32),
        mesh=mesh,
        compiler_params=pltpu.CompilerParams(needs_layout_passes=False),
        scratch_types=[
            pltpu.VMEM((EPW,), jnp.int32),
            pltpu.VMEM((EPW,), jnp.int32),
            pltpu.VMEM((C, D_FEAT), jnp.float32),
            pltpu.VMEM((C, D_FEAT), jnp.float32),
            pltpu.VMEM((C, D_FEAT), jnp.float32),
            pltpu.VMEM((C, D_FEAT), jnp.float32),
            pltpu.VMEM((EPW,), jnp.float32),
            pltpu.SemaphoreType.DMA,
            pltpu.SemaphoreType.DMA,
            pltpu.SemaphoreType.DMA,
            pltpu.SemaphoreType.DMA,
        ],
    )(c_feat, g_feat, src, dst)


def kernel(c_feat, g_feat, edge_index):
    ei = edge_index.astype(jnp.int32)
    out = _decode(c_feat, g_feat, ei[0], ei[1])
    return out.reshape(N_EDGES, 1)


# speedup vs baseline: 5.9743x; 5.1275x over previous
import jax
import jax.numpy as jnp
from jax import lax
from jax.experimental import pallas as pl
from jax.experimental.pallas import tpu as pltpu
from jax.experimental.pallas import tpu_sc as plsc

N_NODES = 10000
N_EDGES = 320000
D_FEAT = 128

NC = 2
NS = 16
NW = NC * NS
EPW = N_EDGES // NW
C = 80
NCHUNK = EPW // C
G = C // 16


def _body(c_hbm, g_hbm, src_hbm, dst_hbm, out_hbm,
          idx_u, idx_v, u0, u1, v0, v1, out_all, trans,
          sem_u0, sem_u1, sem_v0, sem_v1):
    wid = lax.axis_index("s") * NC + lax.axis_index("c")
    base = wid * EPW
    lane = jnp.arange(16, dtype=jnp.int32)

    pltpu.sync_copy(src_hbm.at[pl.ds(base, EPW)], idx_u)
    pltpu.sync_copy(dst_hbm.at[pl.ds(base, EPW)], idx_v)

    def fire(ci, ub, vb, su, sv):
        off = ci * C
        pltpu.async_copy(c_hbm.at[idx_u.at[pl.ds(off, C)]], ub, su)
        pltpu.async_copy(g_hbm.at[idx_v.at[pl.ds(off, C)]], vb, sv)

    def wait(ub, vb, su, sv):
        pltpu.make_async_copy(c_hbm.at[idx_u.at[pl.ds(0, C)]], ub, su).wait()
        pltpu.make_async_copy(g_hbm.at[idx_v.at[pl.ds(0, C)]], vb, sv).wait()

    def compute(ci, ub, vb):
        def group_body(gi, _):
            row0 = gi * 16
            for e in range(16):
                r = row0 + e
                accs = [jnp.zeros((16,), jnp.float32) for _ in range(4)]
                for j in range(D_FEAT // 16):
                    hu = ub[r, pl.ds(j * 16, 16)]
                    hv = vb[r, pl.ds(j * 16, 16)]
                    accs[j % 4] = accs[j % 4] + hu * hv
                trans[e, pl.ds(0, 16)] = (accs[0] + accs[1]) + (accs[2] + accs[3])
            s0 = jnp.zeros((16,), jnp.float32)
            s1 = jnp.zeros((16,), jnp.float32)
            for l in range(16):
                colv = plsc.load_gather(trans, [lane, jnp.full((16,), l, jnp.int32)])
                if l % 2 == 0:
                    s0 = s0 + colv
                else:
                    s1 = s1 + colv
            acc = s0 + s1
            pred = 1.0 / (1.0 + jnp.exp(-acc))
            out_all[pl.ds(ci * C + gi * 16, 16)] = pred
            return 0

        lax.fori_loop(0, G, group_body, 0)

    fire(0, u0, v0, sem_u0, sem_v0)

    def pair_body(i2, _):
        ci0 = i2 * 2
        ci1 = ci0 + 1
        fire(ci1, u1, v1, sem_u1, sem_v1)
        wait(u0, v0, sem_u0, sem_v0)
        compute(ci0, u0, v0)
        fire(ci0 + 2, u0, v0, sem_u0, sem_v0)
        wait(u1, v1, sem_u1, sem_v1)
        compute(ci1, u1, v1)
        return 0

    lax.fori_loop(0, (NCHUNK - 1) // 2, pair_body, 0)
    wait(u0, v0, sem_u0, sem_v0)
    compute(NCHUNK - 1, u0, v0)

    pltpu.sync_copy(out_all, out_hbm.at[pl.ds(base, EPW)])


@jax.jit
def _decode(c_feat, g_feat, src, dst):
    mesh = plsc.VectorSubcoreMesh(core_axis_name="c", subcore_axis_name="s",
                                  num_cores=NC, num_subcores=NS)
    return pl.kernel(
        _body,
        out_type=jax.ShapeDtypeStruct((N_EDGES,), jnp.float32),
        mesh=mesh,
        compiler_params=pltpu.CompilerParams(needs_layout_passes=False),
        scratch_types=[
            pltpu.VMEM((EPW,), jnp.int32),
            pltpu.VMEM((EPW,), jnp.int32),
            pltpu.VMEM((C, D_FEAT), jnp.float32),
            pltpu.VMEM((C, D_FEAT), jnp.float32),
            pltpu.VMEM((C, D_FEAT), jnp.float32),
            pltpu.VMEM((C, D_FEAT), jnp.float32),
            pltpu.VMEM((EPW,), jnp.float32),
            pltpu.VMEM((16, 17), jnp.float32),
            pltpu.SemaphoreType.DMA,
            pltpu.SemaphoreType.DMA,
            pltpu.SemaphoreType.DMA,
            pltpu.SemaphoreType.DMA,
        ],
    )(c_feat, g_feat, src, dst)


def kernel(c_feat, g_feat, edge_index):
    ei = edge_index.astype(jnp.int32)
    out = _decode(c_feat, g_feat, ei[0], ei[1])
    return out.reshape(N_EDGES, 1)
